# baseline (speedup 1.0000x reference)
import jax
import jax.numpy as jnp
from jax import lax
from jax.experimental import pallas as pl
from jax.experimental.pallas import tpu as pltpu
from jax.experimental.pallas import tpu_sc as plsc

_VOCAB = 100000
_EMBED = 100
_MAX_LEN = 20
_BATCH = 128

_INFO = plsc.get_sparse_core_info()
_NC = _INFO.num_cores
_NS = _INFO.num_subcores
_NW = _NC * _NS
_TOK = _BATCH * _MAX_LEN
_TOK_W = _TOK // _NW
_ROWS_W = _BATCH // _NW
_SUB = 7
_REG = _SUB * 16

_mesh = plsc.VectorSubcoreMesh(core_axis_name="c", subcore_axis_name="s")

_SCRATCH = [
    pltpu.VMEM((_TOK_W * _SUB,), jnp.int32),
    pltpu.VMEM((_TOK_W * _SUB, 16), jnp.float32),
    pltpu.VMEM((_TOK_W, 16), jnp.int32),
    pltpu.VMEM((2, _MAX_LEN, _EMBED), jnp.float32),
    pltpu.VMEM((16,), jnp.float32),
    pltpu.VMEM((16,), jnp.float32),
    pltpu.SemaphoreType.DMA,
]


def _sc_body(gidx_hbm, offs_hbm, table_hbm, wt_hbm, b_hbm, out_hbm,
             gidx_v, rows_v, offs_v, wt_v, b_v, out_v, sem):
    wid = lax.axis_index("s") * _NC + lax.axis_index("c")

    pltpu.sync_copy(gidx_hbm.at[pl.ds(wid * _TOK_W * _SUB, _TOK_W * _SUB)],
                    gidx_v)
    gather = pltpu.async_copy(table_hbm.at[gidx_v], rows_v, sem)
    pltpu.sync_copy(offs_hbm.at[pl.ds(wid * _TOK_W, _TOK_W)], offs_v)
    pltpu.sync_copy(wt_hbm, wt_v)
    pltpu.sync_copy(b_hbm, b_v)
    gather.wait()

    lanes = lax.iota(jnp.int32, 16)
    tailmask = jnp.where(lanes >= 12, 1.0, 0.0).astype(jnp.float32)

    def hsum(a):
        for sh in (8, 4, 2, 1):
            a = a + a.at[lanes ^ sh].get(mode="promise_in_bounds")
        return a

    vec = jnp.zeros((16,), jnp.float32)
    for r in range(_ROWS_W):
        def body(t, carry, r=r):
            a0, a1 = carry
            tok = r * _MAX_LEN + t
            off = offs_v[tok, :]
            base = tok * _REG + off + lanes
            for c in range(_SUB):
                start = 16 * c if c < _SUB - 1 else 84
                pos = base + start
                x = plsc.load_gather(rows_v, [pos >> 4, pos & 15])
                if c == _SUB - 1:
                    x = x * tailmask
                a0 = a0 + x * wt_v[0, t, pl.ds(start, 16)]
                a1 = a1 + x * wt_v[1, t, pl.ds(start, 16)]
            return (a0, a1)

        zero = jnp.zeros((16,), jnp.float32)
        a0, a1 = lax.fori_loop(0, _MAX_LEN, body, (zero, zero))
        vec = jnp.where(lanes == 2 * r, hsum(a0), vec)
        vec = jnp.where(lanes == 2 * r + 1, hsum(a1), vec)

    out_v[...] = vec + b_v[...]
    pltpu.sync_copy(out_v, out_hbm.at[pl.ds(wid * 16, 16)])


_sc_forward = pl.kernel(
    _sc_body,
    mesh=_mesh,
    out_type=jax.ShapeDtypeStruct((_NW * 16,), jnp.float32),
    compiler_params=pltpu.CompilerParams(
        use_tc_tiling_on_sc=False, needs_layout_passes=False
    ),
    scratch_types=_SCRATCH,
)


def kernel(input, table, W, b):
    idx = input.reshape(-1).astype(jnp.int32)
    sub = (25 * idx) >> 2
    gidx = (sub[:, None] + jnp.arange(_SUB, dtype=jnp.int32)).reshape(-1)
    offs = jnp.broadcast_to(((idx & 3) << 2)[:, None], (_TOK, 16))
    table16 = table.reshape(_VOCAB * _EMBED // 16, 16)
    wt = W.T.reshape(2, _MAX_LEN, _EMBED)
    b16 = jnp.tile(b.astype(jnp.float32), 8)
    out = _sc_forward(gidx, offs, table16, wt, b16)
    return out.reshape(_NW, 16)[:, : 2 * _ROWS_W].reshape(_BATCH, 2)


# speedup vs baseline: 3.8101x; 3.8101x over previous
import jax
import jax.numpy as jnp
from jax import lax
from jax.experimental import pallas as pl
from jax.experimental.pallas import tpu as pltpu
from jax.experimental.pallas import tpu_sc as plsc

_VOCAB = 100000
_EMBED = 100
_MAX_LEN = 20
_BATCH = 128

_INFO = plsc.get_sparse_core_info()
_NC = _INFO.num_cores
_NS = _INFO.num_subcores
_NW = _NC * _NS
_TOK = _BATCH * _MAX_LEN
_TOK_W = _TOK // _NW
_ROWS_W = _BATCH // _NW

_OFFS = (0, 16, 32, 48, 64, 80, 84)

_mesh = plsc.VectorSubcoreMesh(core_axis_name="c", subcore_axis_name="s")

_SCRATCH = [
    pltpu.VMEM((_TOK_W,), jnp.int32),
    pltpu.VMEM((_TOK_W, _EMBED), jnp.float32),
    pltpu.VMEM((2, _MAX_LEN, _EMBED), jnp.float32),
    pltpu.VMEM((16,), jnp.float32),
    pltpu.VMEM((16,), jnp.float32),
    pltpu.SemaphoreType.DMA,
]


def _sc_body(idx_hbm, table_hbm, wt_hbm, b_hbm, out_hbm,
             idx_v, rows_v, wt_v, b_v, out_v, sem):
    wid = lax.axis_index("s") * _NC + lax.axis_index("c")

    pltpu.sync_copy(idx_hbm.at[pl.ds(wid * _TOK_W, _TOK_W)], idx_v)
    copies = []
    for g in range(_TOK_W // 16):
        idx16 = idx_v[pl.ds(16 * g, 16)]
        for l in range(16):
            row = idx16[l]
            copies.append(pltpu.async_copy(
                table_hbm.at[pl.ds(row, 1), :],
                rows_v.at[pl.ds(16 * g + l, 1), :], sem))
    pltpu.sync_copy(wt_hbm, wt_v)
    pltpu.sync_copy(b_hbm, b_v)
    for cp in copies:
        cp.wait()

    lanes = lax.iota(jnp.int32, 16)
    tailmask = lanes >= 12

    def hsum(a):
        for sh in (8, 4, 2, 1):
            a = a + a.at[lanes ^ sh].get(mode="promise_in_bounds")
        return a

    vec = jnp.zeros((16,), jnp.float32)
    for r in range(_ROWS_W):
        def body(t, carry, r=r):
            a0, a1 = carry
            tok = r * _MAX_LEN + t
            for off in _OFFS:
                x = rows_v[tok, pl.ds(off, 16)]
                if off == _OFFS[-1]:
                    x = jnp.where(tailmask, x, 0.0)
                a0 = a0 + x * wt_v[0, t, pl.ds(off, 16)]
                a1 = a1 + x * wt_v[1, t, pl.ds(off, 16)]
            return (a0, a1)

        zero = jnp.zeros((16,), jnp.float32)
        a0, a1 = lax.fori_loop(0, _MAX_LEN, body, (zero, zero))
        vec = jnp.where(lanes == 2 * r, hsum(a0), vec)
        vec = jnp.where(lanes == 2 * r + 1, hsum(a1), vec)

    out_v[...] = vec + b_v[...]
    pltpu.sync_copy(out_v, out_hbm.at[pl.ds(wid * 16, 16)])


_sc_forward = pl.kernel(
    _sc_body,
    mesh=_mesh,
    out_type=jax.ShapeDtypeStruct((_NW * 16,), jnp.float32),
    scratch_types=_SCRATCH,
)


def kernel(input, table, W, b):
    idx = input.reshape(-1).astype(jnp.int32)
    wt = W.T.reshape(2, _MAX_LEN, _EMBED)
    b16 = jnp.tile(b.astype(jnp.float32), 8)
    out = _sc_forward(idx, table, wt, b16)
    return out.reshape(_NW, 16)[:, : 2 * _ROWS_W].reshape(_BATCH, 2)


# speedup vs baseline: 3.8106x; 1.0002x over previous
import jax
import jax.numpy as jnp
from jax import lax
from jax.experimental import pallas as pl
from jax.experimental.pallas import tpu as pltpu
from jax.experimental.pallas import tpu_sc as plsc

_VOCAB = 100000
_EMBED = 100
_MAX_LEN = 20
_BATCH = 128

_INFO = plsc.get_sparse_core_info()
_NC = _INFO.num_cores
_NS = _INFO.num_subcores
_NW = _NC * _NS
_TOK = _BATCH * _MAX_LEN
_TOK_W = _TOK // _NW
_ROWS_W = _BATCH // _NW

_OFFS = (0, 16, 32, 48, 64, 80, 84)
_WHALF = _MAX_LEN * 128

_mesh = plsc.VectorSubcoreMesh(core_axis_name="c", subcore_axis_name="s")

_SCRATCH = [
    pltpu.VMEM((_TOK_W,), jnp.int32),
    pltpu.VMEM((_TOK_W, _EMBED), jnp.float32),
    pltpu.VMEM((2 * _MAX_LEN * 128,), jnp.float32),
    pltpu.VMEM((16,), jnp.float32),
    pltpu.VMEM((16,), jnp.float32),
    pltpu.SemaphoreType.DMA,
]


def _sc_body(idx_hbm, table_hbm, wt_hbm, b_hbm, out_hbm,
             idx_v, rows_v, wt_v, b_v, out_v, sem):
    wid = lax.axis_index("s") * _NC + lax.axis_index("c")

    pltpu.sync_copy(idx_hbm.at[pl.ds(wid * _TOK_W, _TOK_W)], idx_v)
    copies = []
    for g in range(_TOK_W // 16):
        idx16 = idx_v[pl.ds(16 * g, 16)]
        for l in range(16):
            row = idx16[l]
            copies.append(pltpu.async_copy(
                table_hbm.at[pl.ds(row, 1), :],
                rows_v.at[pl.ds(16 * g + l, 1), :], sem))
    pltpu.sync_copy(wt_hbm, wt_v)
    pltpu.sync_copy(b_hbm, b_v)
    for cp in copies:
        cp.wait()

    lanes = lax.iota(jnp.int32, 16)
    tailmask = lanes >= 12

    def hsum(a):
        for sh in (8, 4, 2, 1):
            a = a + a.at[lanes ^ sh].get(mode="promise_in_bounds")
        return a

    vec = jnp.zeros((16,), jnp.float32)
    for r in range(_ROWS_W):
        def body(t, carry, r=r):
            a0, a1 = carry
            tok = r * _MAX_LEN + t
            for off in _OFFS:
                x = rows_v[tok, pl.ds(off, 16)]
                if off == _OFFS[-1]:
                    x = jnp.where(tailmask, x, 0.0)
                a0 = a0 + x * wt_v[pl.ds(t * 128 + off, 16)]
                a1 = a1 + x * wt_v[pl.ds(_WHALF + t * 128 + off, 16)]
            return (a0, a1)

        zero = jnp.zeros((16,), jnp.float32)
        a0, a1 = lax.fori_loop(0, _MAX_LEN, body, (zero, zero))
        vec = jnp.where(lanes == 2 * r, hsum(a0), vec)
        vec = jnp.where(lanes == 2 * r + 1, hsum(a1), vec)

    out_v[...] = vec + b_v[...]
    pltpu.sync_copy(out_v, out_hbm.at[pl.ds(wid * 16, 16)])


_sc_forward = pl.kernel(
    _sc_body,
    mesh=_mesh,
    out_type=jax.ShapeDtypeStruct((_NW * 16,), jnp.float32),
    scratch_types=_SCRATCH,
)


def kernel(input, table, W, b):
    idx = input.reshape(-1).astype(jnp.int32)
    wt = jnp.pad(W.T.reshape(2, _MAX_LEN, _EMBED),
                 ((0, 0), (0, 0), (0, 28))).reshape(-1)
    b16 = jnp.tile(b.astype(jnp.float32), 8)
    out = _sc_forward(idx, table, wt, b16)
    return out.reshape(_NW, 16)[:, : 2 * _ROWS_W].reshape(_BATCH, 2)
